# baseline (speedup 1.0000x reference)
import functools

import jax
import jax.numpy as jnp
from jax import lax
from jax.experimental import pallas as pl
from jax.experimental.pallas import tpu as pltpu
from jax.experimental.pallas import tpu_sc as plsc

N = 10000
E = 320000
D = 128
DC = 16

NC = 2
NS = 16
NW = NC * NS
EPW = 10240
E2 = EPW * NW
NA = N + 32
B1 = 80
B2 = 128
RPS = N // NS
ZR = 25


def _sc_agg_body(with_cnt, b, feat_hbm, edge_hbm, *refs):
  nchunk = EPW // b
  if with_cnt:
    (out_hbm, outc_hbm, srcv, dstbuf, rows, zbuf, zbufc, ones, acc, accc,
     gsems, ssems, dsems, csems) = refs
  else:
    (out_hbm, srcv, dstbuf, rows, zbuf, acc, gsems, ssems, dsems) = refs
  c = lax.axis_index("c")
  s = lax.axis_index("s")
  w = c * NS + s
  ebase = w * EPW

  zv = jnp.zeros((16,), jnp.float32)

  def zrow(i, carry):
    for col in range(D // 16):
      zbuf[i, pl.ds(col * 16, 16)] = zv
    return carry

  lax.fori_loop(0, ZR, zrow, 0)

  def zslab(i, carry):
    pltpu.sync_copy(zbuf, acc.at[pl.ds(s * RPS + i * ZR, ZR)])
    return carry

  lax.fori_loop(0, RPS // ZR, zslab, 0)

  if with_cnt:
    onev = jnp.where(lax.iota(jnp.int32, 16) == 0, 1.0, 0.0).astype(
        jnp.float32)

    def zrowc(i, carry):
      zbufc[i, :] = zv
      return carry

    lax.fori_loop(0, ZR, zrowc, 0)

    def onerow(i, carry):
      ones[i, :] = onev
      return carry

    lax.fori_loop(0, b, onerow, 0)

    def zslabc(i, carry):
      pltpu.sync_copy(zbufc, accc.at[pl.ds(s * RPS + i * ZR, ZR)])
      return carry

    lax.fori_loop(0, RPS // ZR, zslabc, 0)

  plsc.subcore_barrier()

  pltpu.sync_copy(edge_hbm.at[pl.ds(ebase, EPW)], srcv)

  def gather(j, p):
    return pltpu.make_async_copy(feat_hbm.at[srcv.at[pl.ds(j * b, b)]],
                                 rows.at[p], gsems.at[p])

  def dstload(j, p):
    return pltpu.make_async_copy(edge_hbm.at[pl.ds(E2 + ebase + j * b, b)],
                                 dstbuf.at[p], dsems.at[p])

  def scat(p):
    return pltpu.make_async_copy(rows.at[p], acc.at[dstbuf.at[p]],
                                 ssems.at[p])

  def scatc(p):
    return pltpu.make_async_copy(ones, accc.at[dstbuf.at[p]], csems.at[p])

  dstload(0, 0).start()
  gather(0, 0).start()
  dstload(1, 1).start()
  gather(1, 1).start()
  gather(0, 0).wait()
  dstload(0, 0).wait()
  scat(0).start(add=True)
  if with_cnt:
    scatc(0).start(add=True)

  def step(j, carry):
    p = lax.rem(j, 2)
    q = lax.rem(j + 1, 2)
    scat(q).wait()
    if with_cnt:
      scatc(q).wait()
    dstload(j + 1, q).start()
    gather(j + 1, q).start()
    gather(j, p).wait()
    dstload(j, p).wait()
    scat(p).start(add=True)
    if with_cnt:
      scatc(p).start(add=True)
    return carry

  lax.fori_loop(1, nchunk - 1, step, 0)
  last = nchunk - 1
  lp = last % 2
  lq = (last + 1) % 2
  scat(lq).wait()
  gather(last, lp).wait()
  dstload(last, lp).wait()
  scat(lp).start(add=True)
  scat(lp).wait()
  if with_cnt:
    scatc(lq).wait()
    scatc(lp).start(add=True)
    scatc(lp).wait()
  plsc.subcore_barrier()

  pltpu.sync_copy(acc.at[pl.ds(s * RPS, RPS)],
                  out_hbm.at[c, pl.ds(s * RPS, RPS)])
  if with_cnt:
    pltpu.sync_copy(accc.at[pl.ds(s * RPS, RPS)],
                    outc_hbm.at[c, pl.ds(s * RPS, RPS)])


def _make_sc_agg(with_cnt, b):
  mesh = plsc.VectorSubcoreMesh(core_axis_name="c", subcore_axis_name="s")
  out_type = [jax.ShapeDtypeStruct((NC, N, D), jnp.float32)]
  scratch = [
      pltpu.VMEM((EPW,), jnp.int32),
      pltpu.VMEM((2, b), jnp.int32),
      pltpu.VMEM((2, b, D), jnp.float32),
      pltpu.VMEM((ZR, D), jnp.float32),
  ]
  if with_cnt:
    out_type.append(jax.ShapeDtypeStruct((NC, N, DC), jnp.float32))
    scratch += [
        pltpu.VMEM((ZR, DC), jnp.float32),
        pltpu.VMEM((b, DC), jnp.float32),
    ]
  scratch.append(pltpu.VMEM_SHARED((NA, D), jnp.float32))
  if with_cnt:
    scratch.append(pltpu.VMEM_SHARED((NA, DC), jnp.float32))
  scratch += [pltpu.SemaphoreType.DMA((2,))] * (4 if with_cnt else 3)
  return pl.kernel(
      functools.partial(_sc_agg_body, with_cnt, b),
      out_type=out_type,
      mesh=mesh,
      scratch_types=scratch,
      compiler_params=pltpu.CompilerParams(use_tc_tiling_on_sc=False),
      name=f"sage_sc_agg_{'cnt' if with_cnt else 'plain'}",
  )


_sc_agg_l1 = _make_sc_agg(True, B1)
_sc_agg_l2 = _make_sc_agg(False, B2)


def _tc1_body(p_ref, pc_ref, x_ref, wl_ref, bl_ref, wr_ref, h_ref, ic_ref):
  feats = p_ref[0] + p_ref[1]
  cnt = pc_ref[0][:, 0:1] + pc_ref[1][:, 0:1]
  ic = 1.0 / jnp.maximum(cnt, 1.0)
  mean = feats * ic
  h = (jnp.dot(mean, wl_ref[...], preferred_element_type=jnp.float32)
       + bl_ref[...]
       + jnp.dot(x_ref[...], wr_ref[...], preferred_element_type=jnp.float32))
  h_ref[...] = jnp.maximum(h, 0.0)
  ic_ref[...] = ic


def _tc2_body(p_ref, ic_ref, h1_ref, w2l_ref, b2l_ref, w2r_ref,
              wc1_ref, bc1_ref, wc2_ref, bc2_ref, h2_ref, lg_ref):
  mean = (p_ref[0] + p_ref[1]) * ic_ref[...]
  h2 = (jnp.dot(mean, w2l_ref[...], preferred_element_type=jnp.float32)
        + b2l_ref[...]
        + jnp.dot(h1_ref[...], w2r_ref[...], preferred_element_type=jnp.float32))
  t = jnp.maximum(
      jnp.dot(h2, wc1_ref[...], preferred_element_type=jnp.float32)
      + bc1_ref[...], 0.0)
  lg_ref[...] = (jnp.dot(t, wc2_ref[...], preferred_element_type=jnp.float32)
                 + bc2_ref[...])
  h2_ref[...] = h2


_R = 2000


def _tc1(p1, pc1, x, wl, bl, wr):
  grid = (N // _R,)
  return pl.pallas_call(
      _tc1_body,
      grid=grid,
      in_specs=[
          pl.BlockSpec((NC, _R, D), lambda i: (0, i, 0)),
          pl.BlockSpec((NC, _R, DC), lambda i: (0, i, 0)),
          pl.BlockSpec((_R, D), lambda i: (i, 0)),
          pl.BlockSpec((D, D), lambda i: (0, 0)),
          pl.BlockSpec((1, D), lambda i: (0, 0)),
          pl.BlockSpec((D, D), lambda i: (0, 0)),
      ],
      out_specs=[
          pl.BlockSpec((_R, D), lambda i: (i, 0)),
          pl.BlockSpec((_R, 1), lambda i: (i, 0)),
      ],
      out_shape=[
          jax.ShapeDtypeStruct((N, D), jnp.float32),
          jax.ShapeDtypeStruct((N, 1), jnp.float32),
      ],
      name="sage_tc1",
  )(p1, pc1, x, wl, bl, wr)


def _tc2(p2, ic, h1, w2l, b2l, w2r, wc1, bc1, wc2, bc2):
  grid = (N // _R,)
  return pl.pallas_call(
      _tc2_body,
      grid=grid,
      in_specs=[
          pl.BlockSpec((NC, _R, D), lambda i: (0, i, 0)),
          pl.BlockSpec((_R, 1), lambda i: (i, 0)),
          pl.BlockSpec((_R, D), lambda i: (i, 0)),
          pl.BlockSpec((D, D), lambda i: (0, 0)),
          pl.BlockSpec((1, D), lambda i: (0, 0)),
          pl.BlockSpec((D, D), lambda i: (0, 0)),
          pl.BlockSpec((D, D), lambda i: (0, 0)),
          pl.BlockSpec((1, D), lambda i: (0, 0)),
          pl.BlockSpec((D, 2), lambda i: (0, 0)),
          pl.BlockSpec((1, 2), lambda i: (0, 0)),
      ],
      out_specs=[
          pl.BlockSpec((_R, D), lambda i: (i, 0)),
          pl.BlockSpec((_R, 2), lambda i: (i, 0)),
      ],
      out_shape=[
          jax.ShapeDtypeStruct((N, D), jnp.float32),
          jax.ShapeDtypeStruct((N, 2), jnp.float32),
      ],
      name="sage_tc2",
  )(p2, ic, h1, w2l, b2l, w2r, wc1, bc1, wc2, bc2)


def kernel(x, edge_index, W1l, b1l, W1r, W2l, b2l, W2r, Wc1, bc1, Wc2, bc2):
  pad = EPW - E // NW
  s2 = jnp.pad(edge_index[0].reshape(NW, E // NW), ((0, 0), (0, pad)))
  dummy = jnp.broadcast_to(
      (N + jnp.arange(NW, dtype=jnp.int32))[:, None], (NW, pad))
  d2 = jnp.concatenate([edge_index[1].reshape(NW, E // NW), dummy], axis=1)
  edge_flat = jnp.concatenate([s2.reshape(E2), d2.reshape(E2)])
  p1, pc1 = _sc_agg_l1(x, edge_flat)
  h1, ic = _tc1(p1, pc1, x, W1l.T, b1l.reshape(1, D), W1r.T)
  p2, = _sc_agg_l2(h1, edge_flat)
  h2, logits = _tc2(p2, ic, h1, W2l.T, b2l.reshape(1, D), W2r.T,
                    Wc1.T, bc1.reshape(1, D), Wc2.T, bc2.reshape(1, 2))
  return (h2, logits)


# speedup vs baseline: 1.0008x; 1.0008x over previous
import functools

import jax
import jax.numpy as jnp
from jax import lax
from jax.experimental import pallas as pl
from jax.experimental.pallas import tpu as pltpu
from jax.experimental.pallas import tpu_sc as plsc

N = 10000
E = 320000
D = 128
DC = 16

NC = 2
NS = 16
NW = NC * NS
EPW = 10240
E2 = EPW * NW
NA = N + 240
B1 = 80
B2 = 128
RPS = N // NS
ZR = 25


def _sc_agg_body(with_cnt, b, feat_hbm, edge_hbm, *refs):
  nchunk = EPW // b
  if with_cnt:
    (out_hbm, outc_hbm, srcv, dstbuf, rows, zbuf, zbufc, ones, acc, accc,
     gsems, ssems, dsems, csems) = refs
  else:
    (out_hbm, srcv, dstbuf, rows, zbuf, acc, gsems, ssems, dsems) = refs
  c = lax.axis_index("c")
  s = lax.axis_index("s")
  w = c * NS + s
  ebase = w * EPW

  zv = jnp.zeros((16,), jnp.float32)

  def zrow(i, carry):
    for col in range(D // 16):
      zbuf[i, pl.ds(col * 16, 16)] = zv
    return carry

  lax.fori_loop(0, ZR, zrow, 0)

  def zslab(i, carry):
    pltpu.sync_copy(zbuf, acc.at[pl.ds(s * RPS + i * ZR, ZR)])
    return carry

  lax.fori_loop(0, RPS // ZR, zslab, 0)

  if with_cnt:
    onev = jnp.where(lax.iota(jnp.int32, 16) == 0, 1.0, 0.0).astype(
        jnp.float32)

    def zrowc(i, carry):
      zbufc[i, :] = zv
      return carry

    lax.fori_loop(0, ZR, zrowc, 0)

    def onerow(i, carry):
      ones[i, :] = onev
      return carry

    lax.fori_loop(0, b, onerow, 0)

    def zslabc(i, carry):
      pltpu.sync_copy(zbufc, accc.at[pl.ds(s * RPS + i * ZR, ZR)])
      return carry

    lax.fori_loop(0, RPS // ZR, zslabc, 0)

  plsc.subcore_barrier()

  pltpu.sync_copy(edge_hbm.at[pl.ds(ebase, EPW)], srcv)

  def gather(j, p):
    return pltpu.make_async_copy(feat_hbm.at[srcv.at[pl.ds(j * b, b)]],
                                 rows.at[p], gsems.at[p])

  def dstload(j, p):
    return pltpu.make_async_copy(edge_hbm.at[pl.ds(E2 + ebase + j * b, b)],
                                 dstbuf.at[p], dsems.at[p])

  def scat(p):
    return pltpu.make_async_copy(rows.at[p], acc.at[dstbuf.at[p]],
                                 ssems.at[p])

  def scatc(p):
    return pltpu.make_async_copy(ones, accc.at[dstbuf.at[p]], csems.at[p])

  dstload(0, 0).start()
  gather(0, 0).start()
  dstload(1, 1).start()
  gather(1, 1).start()
  gather(0, 0).wait()
  dstload(0, 0).wait()
  scat(0).start(add=True)
  if with_cnt:
    scatc(0).start(add=True)

  def step(j, carry):
    p = lax.rem(j, 2)
    q = lax.rem(j + 1, 2)
    scat(q).wait()
    if with_cnt:
      scatc(q).wait()
    dstload(j + 1, q).start()
    gather(j + 1, q).start()
    gather(j, p).wait()
    dstload(j, p).wait()
    scat(p).start(add=True)
    if with_cnt:
      scatc(p).start(add=True)
    return carry

  lax.fori_loop(1, nchunk - 1, step, 0)
  last = nchunk - 1
  lp = last % 2
  lq = (last + 1) % 2
  scat(lq).wait()
  gather(last, lp).wait()
  dstload(last, lp).wait()
  scat(lp).start(add=True)
  scat(lp).wait()
  if with_cnt:
    scatc(lq).wait()
    scatc(lp).start(add=True)
    scatc(lp).wait()
  plsc.subcore_barrier()

  pltpu.sync_copy(acc.at[pl.ds(s * RPS, RPS)],
                  out_hbm.at[c, pl.ds(s * RPS, RPS)])
  if with_cnt:
    pltpu.sync_copy(accc.at[pl.ds(s * RPS, RPS)],
                    outc_hbm.at[c, pl.ds(s * RPS, RPS)])


def _make_sc_agg(with_cnt, b):
  mesh = plsc.VectorSubcoreMesh(core_axis_name="c", subcore_axis_name="s")
  out_type = [jax.ShapeDtypeStruct((NC, N, D), jnp.float32)]
  scratch = [
      pltpu.VMEM((EPW,), jnp.int32),
      pltpu.VMEM((2, b), jnp.int32),
      pltpu.VMEM((2, b, D), jnp.float32),
      pltpu.VMEM((ZR, D), jnp.float32),
  ]
  if with_cnt:
    out_type.append(jax.ShapeDtypeStruct((NC, N, DC), jnp.float32))
    scratch += [
        pltpu.VMEM((ZR, DC), jnp.float32),
        pltpu.VMEM((b, DC), jnp.float32),
    ]
  scratch.append(pltpu.VMEM_SHARED((NA, D), jnp.float32))
  if with_cnt:
    scratch.append(pltpu.VMEM_SHARED((NA, DC), jnp.float32))
  scratch += [pltpu.SemaphoreType.DMA((2,))] * (4 if with_cnt else 3)
  return pl.kernel(
      functools.partial(_sc_agg_body, with_cnt, b),
      out_type=out_type,
      mesh=mesh,
      scratch_types=scratch,
      compiler_params=pltpu.CompilerParams(use_tc_tiling_on_sc=False),
      name=f"sage_sc_agg_{'cnt' if with_cnt else 'plain'}",
  )


_sc_agg_l1 = _make_sc_agg(True, B1)
_sc_agg_l2 = _make_sc_agg(False, B2)


def _tc1_body(p_ref, pc_ref, x_ref, wl_ref, bl_ref, wr_ref, h_ref, ic_ref):
  feats = p_ref[0] + p_ref[1]
  cnt = pc_ref[0][:, 0:1] + pc_ref[1][:, 0:1]
  ic = 1.0 / jnp.maximum(cnt, 1.0)
  mean = feats * ic
  h = (jnp.dot(mean, wl_ref[...], preferred_element_type=jnp.float32)
       + bl_ref[...]
       + jnp.dot(x_ref[...], wr_ref[...], preferred_element_type=jnp.float32))
  h_ref[...] = jnp.maximum(h, 0.0)
  ic_ref[...] = ic


def _tc2_body(p_ref, ic_ref, h1_ref, w2l_ref, b2l_ref, w2r_ref,
              wc1_ref, bc1_ref, wc2_ref, bc2_ref, h2_ref, lg_ref):
  mean = (p_ref[0] + p_ref[1]) * ic_ref[...]
  h2 = (jnp.dot(mean, w2l_ref[...], preferred_element_type=jnp.float32)
        + b2l_ref[...]
        + jnp.dot(h1_ref[...], w2r_ref[...], preferred_element_type=jnp.float32))
  t = jnp.maximum(
      jnp.dot(h2, wc1_ref[...], preferred_element_type=jnp.float32)
      + bc1_ref[...], 0.0)
  lg_ref[...] = (jnp.dot(t, wc2_ref[...], preferred_element_type=jnp.float32)
                 + bc2_ref[...])
  h2_ref[...] = h2


_R = 2000


def _tc1(p1, pc1, x, wl, bl, wr):
  grid = (N // _R,)
  return pl.pallas_call(
      _tc1_body,
      grid=grid,
      in_specs=[
          pl.BlockSpec((NC, _R, D), lambda i: (0, i, 0)),
          pl.BlockSpec((NC, _R, DC), lambda i: (0, i, 0)),
          pl.BlockSpec((_R, D), lambda i: (i, 0)),
          pl.BlockSpec((D, D), lambda i: (0, 0)),
          pl.BlockSpec((1, D), lambda i: (0, 0)),
          pl.BlockSpec((D, D), lambda i: (0, 0)),
      ],
      out_specs=[
          pl.BlockSpec((_R, D), lambda i: (i, 0)),
          pl.BlockSpec((_R, 1), lambda i: (i, 0)),
      ],
      out_shape=[
          jax.ShapeDtypeStruct((N, D), jnp.float32),
          jax.ShapeDtypeStruct((N, 1), jnp.float32),
      ],
      name="sage_tc1",
  )(p1, pc1, x, wl, bl, wr)


def _tc2(p2, ic, h1, w2l, b2l, w2r, wc1, bc1, wc2, bc2):
  grid = (N // _R,)
  return pl.pallas_call(
      _tc2_body,
      grid=grid,
      in_specs=[
          pl.BlockSpec((NC, _R, D), lambda i: (0, i, 0)),
          pl.BlockSpec((_R, 1), lambda i: (i, 0)),
          pl.BlockSpec((_R, D), lambda i: (i, 0)),
          pl.BlockSpec((D, D), lambda i: (0, 0)),
          pl.BlockSpec((1, D), lambda i: (0, 0)),
          pl.BlockSpec((D, D), lambda i: (0, 0)),
          pl.BlockSpec((D, D), lambda i: (0, 0)),
          pl.BlockSpec((1, D), lambda i: (0, 0)),
          pl.BlockSpec((D, 2), lambda i: (0, 0)),
          pl.BlockSpec((1, 2), lambda i: (0, 0)),
      ],
      out_specs=[
          pl.BlockSpec((_R, D), lambda i: (i, 0)),
          pl.BlockSpec((_R, 2), lambda i: (i, 0)),
      ],
      out_shape=[
          jax.ShapeDtypeStruct((N, D), jnp.float32),
          jax.ShapeDtypeStruct((N, 2), jnp.float32),
      ],
      name="sage_tc2",
  )(p2, ic, h1, w2l, b2l, w2r, wc1, bc1, wc2, bc2)


def kernel(x, edge_index, W1l, b1l, W1r, W2l, b2l, W2r, Wc1, bc1, Wc2, bc2):
  pad = EPW - E // NW
  s2 = jnp.pad(edge_index[0].reshape(NW, E // NW), ((0, 0), (0, pad)))
  dummy = jnp.broadcast_to(
      (N + jnp.arange(pad, dtype=jnp.int32))[None, :], (NW, pad))
  d2 = jnp.concatenate([edge_index[1].reshape(NW, E // NW), dummy], axis=1)
  edge_flat = jnp.concatenate([s2.reshape(E2), d2.reshape(E2)])
  p1, pc1 = _sc_agg_l1(x, edge_flat)
  h1, ic = _tc1(p1, pc1, x, W1l.T, b1l.reshape(1, D), W1r.T)
  p2, = _sc_agg_l2(h1, edge_flat)
  h2, logits = _tc2(p2, ic, h1, W2l.T, b2l.reshape(1, D), W2r.T,
                    Wc1.T, bc1.reshape(1, D), Wc2.T, bc2.reshape(1, 2))
  return (h2, logits)


# speedup vs baseline: 2.9591x; 2.9569x over previous
import functools

import jax
import jax.numpy as jnp
from jax import lax
from jax.experimental import pallas as pl
from jax.experimental.pallas import tpu as pltpu
from jax.experimental.pallas import tpu_sc as plsc

N = 10000
E = 320000
D = 128
DC = 16

NC = 2
NS = 16
NW = NC * NS
EPW = E // NW
B = 80
NCHUNK = EPW // B
RPS = N // NS
ZR = 25


def _sc_agg_body(with_cnt, feat_hbm, edge_hbm, *refs):
  if with_cnt:
    (out_hbm, outc_hbm, srcv, dstbuf, rows, zbuf, zbufc, ones, acc, accc,
     gsems, ssems, dsems, csems) = refs
  else:
    (out_hbm, srcv, dstbuf, rows, zbuf, acc, gsems, ssems, dsems) = refs
  c = lax.axis_index("c")
  s = lax.axis_index("s")
  w = c * NS + s
  ebase = w * EPW

  zv = jnp.zeros((16,), jnp.float32)

  def zrow(i, carry):
    for col in range(D // 16):
      zbuf[i, pl.ds(col * 16, 16)] = zv
    return carry

  lax.fori_loop(0, ZR, zrow, 0)

  def zslab(i, carry):
    pltpu.sync_copy(zbuf, acc.at[pl.ds(s * RPS + i * ZR, ZR)])
    return carry

  lax.fori_loop(0, RPS // ZR, zslab, 0)

  if with_cnt:
    onev = jnp.where(lax.iota(jnp.int32, 16) == 0, 1.0, 0.0).astype(
        jnp.float32)

    def zrowc(i, carry):
      zbufc[i, :] = zv
      return carry

    lax.fori_loop(0, ZR, zrowc, 0)

    def onerow(i, carry):
      ones[i, :] = onev
      return carry

    lax.fori_loop(0, B, onerow, 0)

    def zslabc(i, carry):
      pltpu.sync_copy(zbufc, accc.at[pl.ds(s * RPS + i * ZR, ZR)])
      return carry

    lax.fori_loop(0, RPS // ZR, zslabc, 0)

  plsc.subcore_barrier()

  pltpu.sync_copy(edge_hbm.at[pl.ds(ebase, EPW)], srcv)

  def gather(j, p):
    return pltpu.make_async_copy(feat_hbm.at[srcv.at[pl.ds(j * B, B)]],
                                 rows.at[p], gsems.at[p])

  def dstload(j, p):
    return pltpu.make_async_copy(edge_hbm.at[pl.ds(E + ebase + j * B, B)],
                                 dstbuf.at[p], dsems.at[p])

  def scat(p):
    return pltpu.make_async_copy(rows.at[p], acc.at[dstbuf.at[p]],
                                 ssems.at[p])

  def scatc(p):
    return pltpu.make_async_copy(ones, accc.at[dstbuf.at[p]], csems.at[p])

  dstload(0, 0).start()
  gather(0, 0).start()
  dstload(1, 1).start()
  gather(1, 1).start()
  gather(0, 0).wait()
  dstload(0, 0).wait()
  scat(0).start(add=True)
  if with_cnt:
    scatc(0).start(add=True)

  def step(j, carry):
    p = lax.rem(j, 2)
    q = lax.rem(j + 1, 2)
    scat(q).wait()
    if with_cnt:
      scatc(q).wait()
    dstload(j + 1, q).start()
    gather(j + 1, q).start()
    gather(j, p).wait()
    dstload(j, p).wait()
    scat(p).start(add=True)
    if with_cnt:
      scatc(p).start(add=True)
    return carry

  lax.fori_loop(1, NCHUNK - 1, step, 0)
  last = NCHUNK - 1
  lp = last % 2
  lq = (last + 1) % 2
  scat(lq).wait()
  gather(last, lp).wait()
  dstload(last, lp).wait()
  scat(lp).start(add=True)
  scat(lp).wait()
  if with_cnt:
    scatc(lq).wait()
    scatc(lp).start(add=True)
    scatc(lp).wait()
  plsc.subcore_barrier()

  pltpu.sync_copy(acc.at[pl.ds(s * RPS, RPS)],
                  out_hbm.at[c, pl.ds(s * RPS, RPS)])
  if with_cnt:
    pltpu.sync_copy(accc.at[pl.ds(s * RPS, RPS)],
                    outc_hbm.at[c, pl.ds(s * RPS, RPS)])


def _make_sc_agg(with_cnt):
  mesh = plsc.VectorSubcoreMesh(core_axis_name="c", subcore_axis_name="s")
  out_type = [jax.ShapeDtypeStruct((NC, N, D), jnp.float32)]
  scratch = [
      pltpu.VMEM((EPW,), jnp.int32),
      pltpu.VMEM((2, B), jnp.int32),
      pltpu.VMEM((2, B, D), jnp.float32),
      pltpu.VMEM((ZR, D), jnp.float32),
  ]
  if with_cnt:
    out_type.append(jax.ShapeDtypeStruct((NC, N, DC), jnp.float32))
    scratch += [
        pltpu.VMEM((ZR, DC), jnp.float32),
        pltpu.VMEM((B, DC), jnp.float32),
    ]
  scratch.append(pltpu.VMEM_SHARED((N, D), jnp.float32))
  if with_cnt:
    scratch.append(pltpu.VMEM_SHARED((N, DC), jnp.float32))
  scratch += [pltpu.SemaphoreType.DMA((2,))] * (4 if with_cnt else 3)
  return pl.kernel(
      functools.partial(_sc_agg_body, with_cnt),
      out_type=out_type,
      mesh=mesh,
      scratch_types=scratch,
      compiler_params=pltpu.CompilerParams(use_tc_tiling_on_sc=False),
      name=f"sage_sc_agg_{'cnt' if with_cnt else 'plain'}",
  )


_sc_agg_l1 = _make_sc_agg(True)
_sc_agg_l2 = _make_sc_agg(False)


def _tc1_body(p_ref, pc_ref, x_ref, wl_ref, bl_ref, wr_ref, h_ref, ic_ref):
  feats = p_ref[0] + p_ref[1]
  cnt = pc_ref[0][:, 0:1] + pc_ref[1][:, 0:1]
  ic = 1.0 / jnp.maximum(cnt, 1.0)
  mean = feats * ic
  h = (jnp.dot(mean, wl_ref[...], preferred_element_type=jnp.float32)
       + bl_ref[...]
       + jnp.dot(x_ref[...], wr_ref[...], preferred_element_type=jnp.float32))
  h_ref[...] = jnp.maximum(h, 0.0)
  ic_ref[...] = ic


def _tc2_body(p_ref, ic_ref, h1_ref, w2l_ref, b2l_ref, w2r_ref,
              wc1_ref, bc1_ref, wc2_ref, bc2_ref, h2_ref, lg_ref):
  mean = (p_ref[0] + p_ref[1]) * ic_ref[...]
  h2 = (jnp.dot(mean, w2l_ref[...], preferred_element_type=jnp.float32)
        + b2l_ref[...]
        + jnp.dot(h1_ref[...], w2r_ref[...], preferred_element_type=jnp.float32))
  t = jnp.maximum(
      jnp.dot(h2, wc1_ref[...], preferred_element_type=jnp.float32)
      + bc1_ref[...], 0.0)
  lg_ref[...] = (jnp.dot(t, wc2_ref[...], preferred_element_type=jnp.float32)
                 + bc2_ref[...])
  h2_ref[...] = h2


_R = 2000


def _tc1(p1, pc1, x, wl, bl, wr):
  grid = (N // _R,)
  return pl.pallas_call(
      _tc1_body,
      grid=grid,
      in_specs=[
          pl.BlockSpec((NC, _R, D), lambda i: (0, i, 0)),
          pl.BlockSpec((NC, _R, DC), lambda i: (0, i, 0)),
          pl.BlockSpec((_R, D), lambda i: (i, 0)),
          pl.BlockSpec((D, D), lambda i: (0, 0)),
          pl.BlockSpec((1, D), lambda i: (0, 0)),
          pl.BlockSpec((D, D), lambda i: (0, 0)),
      ],
      out_specs=[
          pl.BlockSpec((_R, D), lambda i: (i, 0)),
          pl.BlockSpec((_R, 1), lambda i: (i, 0)),
      ],
      out_shape=[
          jax.ShapeDtypeStruct((N, D), jnp.float32),
          jax.ShapeDtypeStruct((N, 1), jnp.float32),
      ],
      name="sage_tc1",
  )(p1, pc1, x, wl, bl, wr)


def _tc2(p2, ic, h1, w2l, b2l, w2r, wc1, bc1, wc2, bc2):
  grid = (N // _R,)
  return pl.pallas_call(
      _tc2_body,
      grid=grid,
      in_specs=[
          pl.BlockSpec((NC, _R, D), lambda i: (0, i, 0)),
          pl.BlockSpec((_R, 1), lambda i: (i, 0)),
          pl.BlockSpec((_R, D), lambda i: (i, 0)),
          pl.BlockSpec((D, D), lambda i: (0, 0)),
          pl.BlockSpec((1, D), lambda i: (0, 0)),
          pl.BlockSpec((D, D), lambda i: (0, 0)),
          pl.BlockSpec((D, D), lambda i: (0, 0)),
          pl.BlockSpec((1, D), lambda i: (0, 0)),
          pl.BlockSpec((D, 2), lambda i: (0, 0)),
          pl.BlockSpec((1, 2), lambda i: (0, 0)),
      ],
      out_specs=[
          pl.BlockSpec((_R, D), lambda i: (i, 0)),
          pl.BlockSpec((_R, 2), lambda i: (i, 0)),
      ],
      out_shape=[
          jax.ShapeDtypeStruct((N, D), jnp.float32),
          jax.ShapeDtypeStruct((N, 2), jnp.float32),
      ],
      name="sage_tc2",
  )(p2, ic, h1, w2l, b2l, w2r, wc1, bc1, wc2, bc2)


def kernel(x, edge_index, W1l, b1l, W1r, W2l, b2l, W2r, Wc1, bc1, Wc2, bc2):
  edge_flat = edge_index.reshape(2 * E)
  p1, pc1 = _sc_agg_l1(x, edge_flat)
  h1, ic = _tc1(p1, pc1, x, W1l.T, b1l.reshape(1, D), W1r.T)
  p2, = _sc_agg_l2(h1, edge_flat)
  h2, logits = _tc2(p2, ic, h1, W2l.T, b2l.reshape(1, D), W2r.T,
                    Wc1.T, bc1.reshape(1, D), Wc2.T, bc2.reshape(1, 2))
  return (h2, logits)
